# baseline (speedup 1.0000x reference)
import functools

import jax
import jax.numpy as jnp
from jax import lax
from jax.experimental import pallas as pl
from jax.experimental.pallas import tpu as pltpu
from jax.experimental.pallas import tpu_sc as plsc

_D = 64
_B = 16384
_NC = 2
_NS = 16
_NW = _NC * _NS
_BPW = _B // _NW
_CHUNK = 128
_NCHUNK = _BPW // _CHUNK

_mesh = plsc.VectorSubcoreMesh(core_axis_name="c", subcore_axis_name="s")


@functools.partial(
    pl.kernel,
    mesh=_mesh,
    out_type=jax.ShapeDtypeStruct((_NW, _D), jnp.float32),
    scratch_types=[
        pltpu.VMEM((_NCHUNK, _CHUNK), jnp.int32),
        pltpu.VMEM((_BPW, _D), jnp.float32),
        pltpu.VMEM((_D,), jnp.float32),
        pltpu.SemaphoreType.DMA,
    ],
    compiler_params=pltpu.CompilerParams(use_tc_tiling_on_sc=False),
)
def _gather_sum(idx_hbm, table_hbm, out_hbm, idx_v, rows_v, acc_v, sem):
    wid = lax.axis_index("s") * _NC + lax.axis_index("c")
    pltpu.sync_copy(idx_hbm.at[wid], idx_v)
    copies = [
        pltpu.async_copy(
            table_hbm.at[idx_v.at[j]],
            rows_v.at[pl.ds(j * _CHUNK, _CHUNK)],
            sem,
        )
        for j in range(_NCHUNK)
    ]
    for c in copies:
        c.wait()

    def body(i, carry):
        a0, a1, a2, a3 = carry
        return (
            a0 + rows_v[i, pl.ds(0, 16)],
            a1 + rows_v[i, pl.ds(16, 16)],
            a2 + rows_v[i, pl.ds(32, 16)],
            a3 + rows_v[i, pl.ds(48, 16)],
        )

    z = jnp.zeros((16,), jnp.float32)
    a0, a1, a2, a3 = lax.fori_loop(0, _BPW, body, (z, z, z, z))
    acc_v[pl.ds(0, 16)] = a0
    acc_v[pl.ds(16, 16)] = a1
    acc_v[pl.ds(32, 16)] = a2
    acc_v[pl.ds(48, 16)] = a3
    pltpu.sync_copy(acc_v, out_hbm.at[wid])


def _tail_body(p_ref, w_ref, b_ref, o_ref):
    tot = jnp.sum(p_ref[...], axis=0, keepdims=True)
    h = 1.0 / (1.0 + jnp.exp(-(tot * (1.0 / _B))))
    logits = jnp.sum(w_ref[...] * h, axis=1, keepdims=True) + b_ref[...]
    s = 1.0 / (1.0 + jnp.exp(-logits))
    row = lax.broadcasted_iota(jnp.int32, (8, 1), 0)
    e = jnp.where(row < 2, jnp.exp(s), 0.0)
    o_ref[...] = e / jnp.sum(e)


def kernel(X, emb, W, b):
    idx = X.astype(jnp.int32).reshape(_NW, _NCHUNK, _CHUNK)
    partials = _gather_sum(idx, emb)
    wp = jnp.zeros((8, _D), jnp.float32).at[:2].set(W)
    bp = jnp.zeros((8, 1), jnp.float32).at[:2, 0].set(b)
    out = pl.pallas_call(
        _tail_body,
        out_shape=jax.ShapeDtypeStruct((8, 1), jnp.float32),
    )(partials, wp, bp)
    return out[:2, 0]


# speedup vs baseline: 1.7161x; 1.7161x over previous
import functools

import jax
import jax.numpy as jnp
from jax import lax
from jax.experimental import pallas as pl
from jax.experimental.pallas import tpu as pltpu
from jax.experimental.pallas import tpu_sc as plsc

_D = 64
_B = 16384
_NC = 2
_NS = 16
_NW = _NC * _NS
_BPW = _B // _NW
_L = 16
_K = 128

_mesh = plsc.VectorSubcoreMesh(core_axis_name="c", subcore_axis_name="s")


@functools.partial(
    pl.kernel,
    mesh=_mesh,
    out_type=jax.ShapeDtypeStruct((_NW, _D), jnp.float32),
    scratch_types=[
        pltpu.VMEM((_BPW,), jnp.int32),
        pltpu.VMEM((_BPW, _D), jnp.float32),
        pltpu.VMEM((_D,), jnp.float32),
        pltpu.SemaphoreType.DMA,
    ],
)
def _gather_sum(idx_hbm, table_hbm, out_hbm, raw_v, rows_v, acc_v, sem):
    wid = lax.axis_index("s") * _NC + lax.axis_index("c")
    pltpu.sync_copy(idx_hbm.at[pl.ds(wid * _BPW, _BPW)], raw_v)

    def fire_chunk(c):
        v = raw_v[pl.ds(c * _L, _L)]
        for r in range(_L):
            pltpu.make_async_copy(
                table_hbm.at[pl.ds(v[r], 1)],
                rows_v.at[pl.ds(c * _L + r, 1)],
                sem,
            ).start()

    def drain_chunk(c):
        for r in range(_L):
            pltpu.make_async_copy(
                table_hbm.at[pl.ds(0, 1)],
                rows_v.at[pl.ds(c * _L + r, 1)],
                sem,
            ).wait()

    nch = _BPW // _L
    ahead = _K // _L

    def fire_only(c, _):
        fire_chunk(c)
        return 0

    def fire_and_drain(c, _):
        drain_chunk(c - ahead)
        fire_chunk(c)
        return 0

    def drain_only(c, _):
        drain_chunk(c)
        return 0

    lax.fori_loop(0, ahead, fire_only, 0)
    lax.fori_loop(ahead, nch, fire_and_drain, 0)
    lax.fori_loop(nch - ahead, nch, drain_only, 0)

    def row_body(i, carry):
        a0, a1, a2, a3 = carry
        return (
            a0 + rows_v[i, pl.ds(0, _L)],
            a1 + rows_v[i, pl.ds(_L, _L)],
            a2 + rows_v[i, pl.ds(2 * _L, _L)],
            a3 + rows_v[i, pl.ds(3 * _L, _L)],
        )

    z = jnp.zeros((_L,), jnp.float32)
    a0, a1, a2, a3 = lax.fori_loop(0, _BPW, row_body, (z, z, z, z))
    acc_v[pl.ds(0, _L)] = a0
    acc_v[pl.ds(_L, _L)] = a1
    acc_v[pl.ds(2 * _L, _L)] = a2
    acc_v[pl.ds(3 * _L, _L)] = a3
    pltpu.sync_copy(acc_v, out_hbm.at[wid])


def _tail_body(p_ref, w_ref, b_ref, o_ref):
    tot = jnp.sum(p_ref[...], axis=0, keepdims=True)
    h = 1.0 / (1.0 + jnp.exp(-(tot * (1.0 / _B))))
    logits = jnp.sum(w_ref[...] * h, axis=1, keepdims=True) + b_ref[...]
    s = 1.0 / (1.0 + jnp.exp(-logits))
    row = lax.broadcasted_iota(jnp.int32, (8, 1), 0)
    e = jnp.where(row < 2, jnp.exp(s), 0.0)
    o_ref[...] = e / jnp.sum(e)


def kernel(X, emb, W, b):
    partials = _gather_sum(X.astype(jnp.int32), emb)
    wp = jnp.zeros((8, _D), jnp.float32).at[:2].set(W)
    bp = jnp.zeros((8, 1), jnp.float32).at[:2, 0].set(b)
    out = pl.pallas_call(
        _tail_body,
        out_shape=jax.ShapeDtypeStruct((8, 1), jnp.float32),
    )(partials, wp, bp)
    return out[:2, 0]
